# baseline (speedup 1.0000x reference)
import functools

import jax
import jax.numpy as jnp
from jax import lax
from jax.experimental import pallas as pl
from jax.experimental.pallas import tpu as pltpu
from jax.experimental.pallas import tpu_sc as plsc

BATCH = 16384
NF = 100
NFIELD = 26
EMB = 16
NC = 2
NS = 16
NW = NC * NS
B_PER_W = BATCH // NW
CB = 16
NCHUNK = B_PER_W // CB
ROWS = CB * NF


def _sc_body(x_hbm, v_hbm, fi_hbm, out_hbm,
             fi_v, e_v, xb0, xb1, ob0, ob1,
             esem, xs0, xs1, os0, os1):
    wid = lax.axis_index("s") * NC + lax.axis_index("c")
    base_row = wid * (B_PER_W * NF)

    pltpu.sync_copy(fi_hbm, fi_v)
    pltpu.make_async_copy(v_hbm.at[fi_v], e_v, esem).start()
    pltpu.make_async_copy(v_hbm.at[fi_v], e_v, esem).wait()

    xbufs = (xb0, xb1)
    xsems = (xs0, xs1)
    obufs = (ob0, ob1)
    osems = (os0, os1)

    def start_x(c, slot):
        off = base_row + c * ROWS
        pltpu.make_async_copy(
            x_hbm.at[pl.ds(off, ROWS)], xbufs[slot], xsems[slot]).start()

    start_x(0, 0)
    start_x(1, 1)

    starts = (0, 16, 32, 48, 64, 80, 84)
    dnums = lax.GatherDimensionNumbers(
        offset_dims=(), collapsed_slice_dims=(0,), start_index_map=(0,))

    def splat(vec, j):
        idx = jnp.full((16, 1), j, jnp.int32)
        return lax.gather(vec, idx, dnums, (1,),
                          mode=lax.GatherScatterMode.PROMISE_IN_BOUNDS)

    def compute(xbuf, obuf):
        @pl.loop(0, CB)
        def _b(bl):
            base = bl * NF
            for s in starts:
                xv = xbuf[pl.ds(base + s, 16)]
                for j in range(16):
                    f = s + j
                    obuf[pl.ds((base + f) * EMB, EMB)] = splat(xv, j) * e_v[f, :EMB]

    @pl.loop(0, NCHUNK, step=2)
    def _c(c0):
        for s in range(2):
            c = c0 + s
            pltpu.make_async_copy(
                x_hbm.at[pl.ds(0, ROWS)], xbufs[s], xsems[s]).wait()

            @pl.when(c >= 2)
            def _wait_prev():
                pltpu.make_async_copy(
                    obufs[s], out_hbm.at[pl.ds(0, ROWS * EMB)], osems[s]).wait()

            compute(xbufs[s], obufs[s])
            pltpu.make_async_copy(
                obufs[s],
                out_hbm.at[pl.ds((base_row + c * ROWS) * EMB, ROWS * EMB)],
                osems[s]).start()

            @pl.when(c + 2 < NCHUNK)
            def _prefetch():
                start_x(c + 2, s)

    for s in range(2):
        pltpu.make_async_copy(
            obufs[s], out_hbm.at[pl.ds(0, ROWS * EMB)], osems[s]).wait()


_sc_kernel = functools.partial(
    pl.kernel,
    out_type=jax.ShapeDtypeStruct((BATCH * NF * EMB,), jnp.float32),
    mesh=plsc.VectorSubcoreMesh(core_axis_name="c", subcore_axis_name="s"),
    scratch_types=[
        pltpu.VMEM((NF,), jnp.int32),
        pltpu.VMEM((NF, 128), jnp.float32),
        pltpu.VMEM((ROWS,), jnp.float32),
        pltpu.VMEM((ROWS,), jnp.float32),
        pltpu.VMEM((ROWS * EMB,), jnp.float32),
        pltpu.VMEM((ROWS * EMB,), jnp.float32),
        pltpu.SemaphoreType.DMA,
        pltpu.SemaphoreType.DMA,
        pltpu.SemaphoreType.DMA,
        pltpu.SemaphoreType.DMA,
        pltpu.SemaphoreType.DMA,
    ],
)(_sc_body)


def kernel(inputs, V, field_index):
    v_pad = jnp.pad(V, ((0, 0), (0, 128 - EMB)))
    out2 = _sc_kernel(inputs.reshape(BATCH * NF), v_pad, field_index)
    return out2.reshape(BATCH, NF, EMB)


# speedup vs baseline: 32.1378x; 32.1378x over previous
import jax
import jax.numpy as jnp
from jax import lax
from jax.experimental import pallas as pl
from jax.experimental.pallas import tpu as pltpu

BATCH = 16384
NF = 100
NFIELD = 26
EMB = 16
B_CH = 1024
GRID = BATCH // B_CH


def _body(fi_ref, vt_ref, x_ref, out_ref, et_ref):
    @pl.when(pl.program_id(0) == 0)
    def _build_et():
        c_iota = lax.broadcasted_iota(jnp.int32, (NFIELD, NF), 0).astype(jnp.float32)
        onehot = (c_iota == jnp.broadcast_to(fi_ref[...], (NFIELD, NF)))
        et_ref[...] = lax.dot(vt_ref[...], onehot.astype(jnp.float32),
                              preferred_element_type=jnp.float32)

    for f in range(NF):
        x_row = x_ref[f:f + 1, :]
        e_col = et_ref[:, f:f + 1]
        out_ref[f] = e_col * x_row


def kernel(inputs, V, field_index):
    x_t = inputs.T
    v_t = V.T
    fi_f = field_index.astype(jnp.float32).reshape(1, NF)
    out_t = pl.pallas_call(
        _body,
        grid=(GRID,),
        in_specs=[
            pl.BlockSpec((1, NF), lambda i: (0, 0)),
            pl.BlockSpec((EMB, NFIELD), lambda i: (0, 0)),
            pl.BlockSpec((NF, B_CH), lambda i: (0, i)),
        ],
        out_specs=pl.BlockSpec((NF, EMB, B_CH), lambda i: (0, 0, i)),
        out_shape=jax.ShapeDtypeStruct((NF, EMB, BATCH), jnp.float32),
        scratch_shapes=[pltpu.VMEM((EMB, NF), jnp.float32)],
        compiler_params=pltpu.CompilerParams(
            dimension_semantics=("arbitrary",),
        ),
    )(fi_f, v_t, x_t)
    return jnp.transpose(out_t, (2, 0, 1))
